# baseline (speedup 1.0000x reference)
import functools

import jax
import jax.numpy as jnp
from jax import lax
from jax.experimental import pallas as pl
from jax.experimental.pallas import tpu as pltpu
from jax.experimental.pallas import tpu_sc as plsc

B = 16384
D_USER = 32
NC, NS = 2, 16
NW = NC * NS
B_PER_W = B // NW
GRP = 16


def _gather_user_cols(user_id_i32, user_table_t):
    mesh = plsc.VectorSubcoreMesh(core_axis_name="c", subcore_axis_name="s")

    @functools.partial(
        pl.kernel,
        mesh=mesh,
        out_type=jax.ShapeDtypeStruct((D_USER, B), jnp.float32),
        scratch_types=[
            pltpu.VMEM((B_PER_W,), jnp.int32),
            pltpu.VMEM((GRP, D_USER, 128), jnp.float32),
            pltpu.VMEM((D_USER, B_PER_W), jnp.float32),
            pltpu.SemaphoreType.DMA,
        ],
        compiler_params=pltpu.CompilerParams(needs_layout_passes=False),
    )
    def gather_k(idx_hbm, tbl_hbm, out_hbm, idx_v, blk_v, cols_v, sem):
        wid = lax.axis_index("s") * NC + lax.axis_index("c")
        base = wid * B_PER_W

        pltpu.sync_copy(idx_hbm.at[pl.ds(base, B_PER_W)], idx_v)
        d_lo = lax.iota(jnp.int32, 16)
        d_hi = d_lo + 16

        def body(j, carry):
            vec = idx_v[pl.ds(j * GRP, GRP)]
            lanes = vec & 127
            handles = []
            for t in range(GRP):
                blk = (vec[t] >> 7) * 128
                handles.append(pltpu.async_copy(
                    tbl_hbm.at[:, pl.ds(blk, 128)], blk_v.at[t], sem))

            def gather_one(t):
                lane = jnp.broadcast_to(lanes[t], (16,))
                col = jnp.broadcast_to(j * GRP + t, (16,))
                lo = plsc.load_gather(blk_v.at[t], [d_lo, lane])
                hi = plsc.load_gather(blk_v.at[t], [d_hi, lane])
                plsc.store_scatter(cols_v, [d_lo, col], lo)
                plsc.store_scatter(cols_v, [d_hi, col], hi)

            for t in range(GRP // 2):
                handles[t].wait()
            for t in range(GRP // 2):
                gather_one(t)
            for t in range(GRP // 2, GRP):
                handles[t].wait()
            for t in range(GRP // 2, GRP):
                gather_one(t)
            return carry

        lax.fori_loop(0, B_PER_W // GRP, body, 0)
        pltpu.sync_copy(cols_v, out_hbm.at[:, pl.ds(base, B_PER_W)])

    return gather_k(user_id_i32, user_table_t)


def _mlp_body(eut, age, gen, occ, at, gt, ot, w1u, w1a, w1g, w1o,
              b1, w2, b2, outt):
    f32 = jnp.float32
    a_proj = jnp.dot(at[...], w1a[...], preferred_element_type=f32)
    g_proj = jnp.dot(gt[...], w1g[...], preferred_element_type=f32)
    o_proj = jnp.dot(ot[...], w1o[...], preferred_element_type=f32)
    oh_a = (age[...] == lax.broadcasted_iota(jnp.int32, (1, 8), 1)).astype(f32)
    oh_g = (gen[...] == lax.broadcasted_iota(jnp.int32, (1, 3), 1)).astype(f32)
    oh_o = (occ[...] == lax.broadcasted_iota(jnp.int32, (1, 22), 1)).astype(f32)
    h = lax.dot_general(eut[...], w1u[...], (((0,), (0,)), ((), ())),
                        preferred_element_type=f32)
    h = h + jnp.dot(oh_a, a_proj, preferred_element_type=f32)
    h = h + jnp.dot(oh_g, g_proj, preferred_element_type=f32)
    h = h + jnp.dot(oh_o, o_proj, preferred_element_type=f32)
    h = jnp.maximum(h + b1[...], 0.0)
    outt[...] = lax.dot_general(w2[...], h, (((0,), (1,)), ((), ())),
                                preferred_element_type=f32) + b2[...]


def kernel(user_id, user_age_binned, user_gender, user_occupation,
           user_table, age_table, gender_table, occ_table,
           W1, b1, W2, b2):
    eut = _gather_user_cols(user_id.astype(jnp.int32), user_table.T)

    BB = 2048
    grid = B // BB
    age2 = user_age_binned.astype(jnp.int32).reshape(B, 1)
    gen2 = user_gender.astype(jnp.int32).reshape(B, 1)
    occ2 = user_occupation.astype(jnp.int32).reshape(B, 1)
    w1u = W1[0:32, :]
    w1a = W1[32:36, :]
    w1g = W1[36:37, :]
    w1o = W1[37:45, :]
    b1r = b1.reshape(1, -1)
    b2c = b2.reshape(-1, 1)

    rep = lambda shape: pl.BlockSpec(shape, lambda i: tuple(0 for _ in shape))
    outt = pl.pallas_call(
        _mlp_body,
        grid=(grid,),
        in_specs=[
            pl.BlockSpec((D_USER, BB), lambda i: (0, i)),
            pl.BlockSpec((BB, 1), lambda i: (i, 0)),
            pl.BlockSpec((BB, 1), lambda i: (i, 0)),
            pl.BlockSpec((BB, 1), lambda i: (i, 0)),
            rep((8, 4)),
            rep((3, 1)),
            rep((22, 8)),
            rep((32, 64)),
            rep((4, 64)),
            rep((1, 64)),
            rep((8, 64)),
            rep((1, 64)),
            rep((64, 32)),
            rep((32, 1)),
        ],
        out_specs=pl.BlockSpec((32, BB), lambda i: (0, i)),
        out_shape=jax.ShapeDtypeStruct((32, B), jnp.float32),
    )(eut, age2, gen2, occ2, age_table, gender_table, occ_table,
      w1u, w1a, w1g, w1o, b1r, W2, b2c)
    return outt.T


# speedup vs baseline: 1.1797x; 1.1797x over previous
import functools

import jax
import jax.numpy as jnp
from jax import lax
from jax.experimental import pallas as pl
from jax.experimental.pallas import tpu as pltpu
from jax.experimental.pallas import tpu_sc as plsc

B = 16384
D_USER = 32
NC, NS = 2, 16
NW = NC * NS
B_PER_W = B // NW
GRP = 16


def _gather_user_cols(user_id_i32, user_table_t):
    mesh = plsc.VectorSubcoreMesh(core_axis_name="c", subcore_axis_name="s")

    @functools.partial(
        pl.kernel,
        mesh=mesh,
        out_type=jax.ShapeDtypeStruct((D_USER, B), jnp.float32),
        scratch_types=[
            pltpu.VMEM((B_PER_W,), jnp.int32),
            pltpu.VMEM((GRP, D_USER, 128), jnp.float32),
            pltpu.VMEM((D_USER, B_PER_W), jnp.float32),
            pltpu.SemaphoreType.DMA,
        ],
        compiler_params=pltpu.CompilerParams(needs_layout_passes=False),
    )
    def gather_k(idx_hbm, tbl_hbm, out_hbm, idx_v, blk_v, cols_v, sem):
        wid = lax.axis_index("s") * NC + lax.axis_index("c")
        base = wid * B_PER_W

        pltpu.sync_copy(idx_hbm.at[pl.ds(base, B_PER_W)], idx_v)
        d_lo = lax.iota(jnp.int32, 16)
        d_hi = d_lo + 16
        n_grp = B_PER_W // GRP

        def issue(vec, t):
            blk = (vec[t] >> 7) * 128
            pltpu.async_copy(tbl_hbm.at[:, pl.ds(blk, 128)],
                             blk_v.at[t], sem)

        def gather_one(vec, j, t):
            lane = jnp.broadcast_to(vec[t] & 127, (16,))
            col = jnp.broadcast_to(j * GRP + t, (16,))
            lo = plsc.load_gather(blk_v.at[t], [d_lo, lane])
            hi = plsc.load_gather(blk_v.at[t], [d_hi, lane])
            plsc.store_scatter(cols_v, [d_lo, col], lo)
            plsc.store_scatter(cols_v, [d_hi, col], hi)

        vec0 = idx_v[pl.ds(0, GRP)]
        for t in range(GRP):
            issue(vec0, t)

        def body(j, carry):
            vec = idx_v[pl.ds(j * GRP, GRP)]
            nxt = idx_v[pl.ds(jnp.minimum(j + 1, n_grp - 1) * GRP, GRP)]
            for t in range(GRP):
                pltpu.make_async_copy(tbl_hbm.at[:, pl.ds(0, 128)],
                                      blk_v.at[t], sem).wait()
                gather_one(vec, j, t)
                issue(nxt, t)
            return carry

        lax.fori_loop(0, n_grp, body, 0)
        vec_l = idx_v[pl.ds((n_grp - 1) * GRP, GRP)]
        for t in range(GRP):
            pltpu.make_async_copy(tbl_hbm.at[:, pl.ds(0, 128)],
                                  blk_v.at[t], sem).wait()
            gather_one(vec_l, n_grp - 1, t)
        pltpu.sync_copy(cols_v, out_hbm.at[:, pl.ds(base, B_PER_W)])

    return gather_k(user_id_i32, user_table_t)


def _mlp_body(eut, age, gen, occ, at, gt, ot, w1u, w1a, w1g, w1o,
              b1, w2, b2, outt):
    f32 = jnp.float32
    a_proj = jnp.dot(at[...], w1a[...], preferred_element_type=f32)
    g_proj = jnp.dot(gt[...], w1g[...], preferred_element_type=f32)
    o_proj = jnp.dot(ot[...], w1o[...], preferred_element_type=f32)
    oh_a = (age[...] == lax.broadcasted_iota(jnp.int32, (1, 8), 1)).astype(f32)
    oh_g = (gen[...] == lax.broadcasted_iota(jnp.int32, (1, 3), 1)).astype(f32)
    oh_o = (occ[...] == lax.broadcasted_iota(jnp.int32, (1, 22), 1)).astype(f32)
    h = lax.dot_general(eut[...], w1u[...], (((0,), (0,)), ((), ())),
                        preferred_element_type=f32)
    h = h + jnp.dot(oh_a, a_proj, preferred_element_type=f32)
    h = h + jnp.dot(oh_g, g_proj, preferred_element_type=f32)
    h = h + jnp.dot(oh_o, o_proj, preferred_element_type=f32)
    h = jnp.maximum(h + b1[...], 0.0)
    outt[...] = lax.dot_general(w2[...], h, (((0,), (1,)), ((), ())),
                                preferred_element_type=f32) + b2[...]


def kernel(user_id, user_age_binned, user_gender, user_occupation,
           user_table, age_table, gender_table, occ_table,
           W1, b1, W2, b2):
    eut = _gather_user_cols(user_id.astype(jnp.int32), user_table.T)

    BB = 2048
    grid = B // BB
    age2 = user_age_binned.astype(jnp.int32).reshape(B, 1)
    gen2 = user_gender.astype(jnp.int32).reshape(B, 1)
    occ2 = user_occupation.astype(jnp.int32).reshape(B, 1)
    w1u = W1[0:32, :]
    w1a = W1[32:36, :]
    w1g = W1[36:37, :]
    w1o = W1[37:45, :]
    b1r = b1.reshape(1, -1)
    b2c = b2.reshape(-1, 1)

    rep = lambda shape: pl.BlockSpec(shape, lambda i: tuple(0 for _ in shape))
    outt = pl.pallas_call(
        _mlp_body,
        grid=(grid,),
        in_specs=[
            pl.BlockSpec((D_USER, BB), lambda i: (0, i)),
            pl.BlockSpec((BB, 1), lambda i: (i, 0)),
            pl.BlockSpec((BB, 1), lambda i: (i, 0)),
            pl.BlockSpec((BB, 1), lambda i: (i, 0)),
            rep((8, 4)),
            rep((3, 1)),
            rep((22, 8)),
            rep((32, 64)),
            rep((4, 64)),
            rep((1, 64)),
            rep((8, 64)),
            rep((1, 64)),
            rep((64, 32)),
            rep((32, 1)),
        ],
        out_specs=pl.BlockSpec((32, BB), lambda i: (0, i)),
        out_shape=jax.ShapeDtypeStruct((32, B), jnp.float32),
    )(eut, age2, gen2, occ2, age_table, gender_table, occ_table,
      w1u, w1a, w1g, w1o, b1r, W2, b2c)
    return outt.T
